# Initial kernel scaffold; baseline (speedup 1.0000x reference)
import jax
import jax.numpy as jnp
from jax.experimental import pallas as pl


def kernel(data, W0, b0, gamma0, beta0, p0, W1, b1, gamma1, beta1, p1):
    raise NotImplementedError("write your pallas kernel here")


# speedup vs baseline: 2129.6060x; 2129.6060x over previous
import jax
import jax.numpy as jnp
from jax.experimental import pallas as pl

_B = 32
_N0 = 128
_HID = 128
_BN_EPS = 1e-5


def _body(data_ref, w0_ref, b0_ref, g0_ref, be0_ref, p0_ref,
          w1_ref, b1_ref, g1_ref, be1_ref, p1_ref, out_ref):
    data = data_ref[:, :]
    rs = jax.lax.rsqrt(jnp.float32(_N0))
    m = jnp.sum(data, axis=1, keepdims=True) * (rs * rs)

    inv_bn = 1.0 / jnp.sqrt(jnp.float32(1.0 + _BN_EPS))

    h0 = m * w0_ref[0:1, :] + b0_ref[0:1, :]
    h0 = jnp.maximum(h0, 0.0)
    h0 = h0 * (g0_ref[0:1, :] * inv_bn) + be0_ref[0:1, :]
    p0 = p0_ref[0:1, :]
    p0n = p0 * jax.lax.rsqrt(jnp.sum(p0 * p0))
    s0 = jnp.sum(h0 * p0n, axis=1, keepdims=True)
    x1 = h0 * jnp.tanh(s0)

    h1 = jnp.dot(x1, w1_ref[:, :], preferred_element_type=jnp.float32)
    h1 = h1 + b1_ref[0:1, :]
    h1 = jnp.maximum(h1, 0.0)
    h1 = h1 * (g1_ref[0:1, :] * inv_bn) + be1_ref[0:1, :]
    p1 = p1_ref[0:1, :]
    p1n = p1 * jax.lax.rsqrt(jnp.sum(p1 * p1))
    s1 = jnp.sum(h1 * p1n, axis=1, keepdims=True)
    out_ref[:, :] = h1 * jnp.tanh(s1)


def kernel(data, W0, b0, gamma0, beta0, p0, W1, b1, gamma1, beta1, p1):
    args = (data, W0.reshape(1, _HID), b0.reshape(1, _HID),
            gamma0.reshape(1, _HID), beta0.reshape(1, _HID),
            p0.reshape(1, _HID), W1,
            b1.reshape(1, _HID), gamma1.reshape(1, _HID),
            beta1.reshape(1, _HID), p1.reshape(1, _HID))
    return pl.pallas_call(
        _body,
        out_shape=jax.ShapeDtypeStruct((_B, _HID), jnp.float32),
    )(*args)
